# Initial kernel scaffold; baseline (speedup 1.0000x reference)
import jax
import jax.numpy as jnp
from jax.experimental import pallas as pl


def kernel(x, Wg, bg, W1, b1, g1, be1, W2, b2, g2, be2):
    raise NotImplementedError("write your pallas kernel here")


# speedup vs baseline: 2.5989x; 2.5989x over previous
import functools

import jax
import jax.numpy as jnp
from jax.experimental import pallas as pl
from jax.experimental.pallas import tpu as pltpu

_NEG = -1e30
_POS = 1e30


def _layer_norm(h, g, b, eps=1e-5):
    mu = jnp.mean(h, axis=-1, keepdims=True)
    var = jnp.mean((h - mu) ** 2, axis=-1, keepdims=True)
    return (h - mu) * jax.lax.rsqrt(var + eps) * g + b


def _pick_extreme(s, iota, largest):
    if largest:
        m = jnp.max(s, axis=-1, keepdims=True)
    else:
        m = jnp.min(s, axis=-1, keepdims=True)
    eq = s == m
    idx = jnp.min(jnp.where(eq, iota, s.shape[-1]), axis=-1, keepdims=True)
    return iota == idx, m


def _moe_body(E, BT,
              x_ref, wg_ref, bg_ref, w1_ref, b1_ref, g1_ref, be1_ref,
              w2_ref, b2_ref, g2_ref, be2_ref,
              out_ref, top_ref, bot_ref, ss_ref,
              acc_top, acc_bot, wt_s, wb_s):
    e = pl.program_id(1)

    @pl.when(e == 0)
    def _gate():
        x = x_ref[...]
        s = jax.lax.dot_general(
            x, wg_ref[...], (((1,), (1,)), ((), ())),
            preferred_element_type=jnp.float32) + bg_ref[...]
        iota = jax.lax.broadcasted_iota(jnp.int32, s.shape, 1)
        m1, s1 = _pick_extreme(s, iota, True)
        s_m = jnp.where(m1, _NEG, s)
        m2, s2 = _pick_extreme(s_m, iota, True)
        e2 = jnp.exp(s2 - s1)
        z = 1.0 + e2
        wt_s[...] = jnp.where(m1, 1.0 / z, 0.0) + jnp.where(m2, e2 / z, 0.0)
        q1, n1 = _pick_extreme(s, iota, False)
        s_q = jnp.where(q1, _POS, s)
        q2, n2 = _pick_extreme(s_q, iota, False)
        eb = jnp.exp(n1 - n2)
        zb = 1.0 + eb
        wb_s[...] = jnp.where(q1, eb / zb, 0.0) + jnp.where(q2, 1.0 / zb, 0.0)
        acc_top[...] = jnp.zeros_like(acc_top)
        acc_bot[...] = jnp.zeros_like(acc_bot)

    x = x_ref[...]
    h = jax.lax.dot_general(
        x, w1_ref[0], (((1,), (1,)), ((), ())),
        preferred_element_type=jnp.float32) + b1_ref[0]
    h = _layer_norm(h, g1_ref[0], be1_ref[0])
    h = jnp.maximum(h, 0.0)
    o = jax.lax.dot_general(
        h, w2_ref[0], (((1,), (1,)), ((), ())),
        preferred_element_type=jnp.float32) + b2_ref[0]
    o = _layer_norm(o, g2_ref[0], be2_ref[0])

    lane = jax.lax.broadcasted_iota(jnp.int32, (BT, E), 1)
    sel = lane == e
    wt_col = jnp.sum(jnp.where(sel, wt_s[...], 0.0), axis=1, keepdims=True)
    wb_col = jnp.sum(jnp.where(sel, wb_s[...], 0.0), axis=1, keepdims=True)
    acc_top[...] += wt_col * o
    acc_bot[...] += wb_col * o

    @pl.when(e == E - 1)
    def _emit():
        at = acc_top[...]
        ab = acc_bot[...]
        top_ref[...] = at
        bot_ref[...] = ab
        out_ref[...] = at + x_ref[...]
        d = at - ab
        ss_ref[...] = jnp.full(ss_ref.shape, jnp.sum(d * d), jnp.float32)


def _moe_fused(xf, Wg, bg, W1, b1, g1, be1, W2, b2, g2, be2, *, BT):
    T, D = xf.shape
    E = Wg.shape[0]
    ntb = T // BT
    grid = (ntb, E)

    b1r = b1.reshape(E, 1, D)
    g1r = g1.reshape(E, 1, D)
    be1r = be1.reshape(E, 1, D)
    b2r = b2.reshape(E, 1, D)
    g2r = g2.reshape(E, 1, D)
    be2r = be2.reshape(E, 1, D)
    bgr = bg.reshape(1, E)

    def tb_map(tb, e):
        return (tb, 0)

    def e3_map(tb, e):
        return (e, 0, 0)

    out, top, bot, ss = pl.pallas_call(
        functools.partial(_moe_body, E, BT),
        grid=grid,
        in_specs=[
            pl.BlockSpec((BT, D), tb_map),
            pl.BlockSpec((E, D), lambda tb, e: (0, 0)),
            pl.BlockSpec((1, E), lambda tb, e: (0, 0)),
            pl.BlockSpec((1, D, D), e3_map),
            pl.BlockSpec((1, 1, D), e3_map),
            pl.BlockSpec((1, 1, D), e3_map),
            pl.BlockSpec((1, 1, D), e3_map),
            pl.BlockSpec((1, D, D), e3_map),
            pl.BlockSpec((1, 1, D), e3_map),
            pl.BlockSpec((1, 1, D), e3_map),
            pl.BlockSpec((1, 1, D), e3_map),
        ],
        out_specs=[
            pl.BlockSpec((BT, D), tb_map),
            pl.BlockSpec((BT, D), tb_map),
            pl.BlockSpec((BT, D), tb_map),
            pl.BlockSpec((8, 128), tb_map),
        ],
        out_shape=[
            jax.ShapeDtypeStruct((T, D), jnp.float32),
            jax.ShapeDtypeStruct((T, D), jnp.float32),
            jax.ShapeDtypeStruct((T, D), jnp.float32),
            jax.ShapeDtypeStruct((ntb * 8, 128), jnp.float32),
        ],
        scratch_shapes=[
            pltpu.VMEM((BT, D), jnp.float32),
            pltpu.VMEM((BT, D), jnp.float32),
            pltpu.VMEM((BT, E), jnp.float32),
            pltpu.VMEM((BT, E), jnp.float32),
        ],
    )(xf, Wg, bgr, W1, b1r, g1r, be1r, W2, b2r, g2r, be2r)
    return out, top, bot, ss


def kernel(x, Wg, bg, W1, b1, g1, be1, W2, b2, g2, be2):
    B_, N_, D_ = x.shape
    T = B_ * N_
    xf = x.reshape(T, D_)
    BT = min(512, T)
    out, top, bot, ss = _moe_fused(
        xf, Wg, bg, W1, b1, g1, be1, W2, b2, g2, be2, BT=BT)
    total_ss = jnp.sum(ss[::8, 0])
    dist = jnp.sqrt(total_ss)
    loss = jnp.mean(1.0 / (dist + 1e-8))
    return (out.reshape(B_, N_, D_),
            top.reshape(B_, N_, D_),
            bot.reshape(B_, N_, D_),
            loss)


# speedup vs baseline: 2.7816x; 1.0703x over previous
import functools

import jax
import jax.numpy as jnp
from jax.experimental import pallas as pl
from jax.experimental.pallas import tpu as pltpu

_NEG = -1e30
_POS = 1e30


def _layer_norm(h, g, b, eps=1e-5):
    mu = jnp.mean(h, axis=-1, keepdims=True)
    var = jnp.mean((h - mu) ** 2, axis=-1, keepdims=True)
    return (h - mu) * jax.lax.rsqrt(var + eps) * g + b


def _pick_extreme(s, iota, largest):
    if largest:
        m = jnp.max(s, axis=-1, keepdims=True)
    else:
        m = jnp.min(s, axis=-1, keepdims=True)
    eq = s == m
    idx = jnp.min(jnp.where(eq, iota, s.shape[-1]), axis=-1, keepdims=True)
    return iota == idx, m


def _moe_body(E, BT,
              x_ref, wg_ref, bg_ref, w1_ref, b1_ref, g1_ref, be1_ref,
              w2_ref, b2_ref, g2_ref, be2_ref,
              out_ref, top_ref, bot_ref, ss_ref,
              wt_s, wb_s):
    e = pl.program_id(1)

    @pl.when(e == 0)
    def _gate():
        x = x_ref[...]
        s = jax.lax.dot_general(
            x, wg_ref[...], (((1,), (1,)), ((), ())),
            preferred_element_type=jnp.float32) + bg_ref[...]
        iota = jax.lax.broadcasted_iota(jnp.int32, s.shape, 1)
        m1, s1 = _pick_extreme(s, iota, True)
        s_m = jnp.where(m1, _NEG, s)
        m2, s2 = _pick_extreme(s_m, iota, True)
        e2 = jnp.exp(s2 - s1)
        z = 1.0 + e2
        wt_s[...] = jnp.where(m1, 1.0 / z, 0.0) + jnp.where(m2, e2 / z, 0.0)
        q1, n1 = _pick_extreme(s, iota, False)
        s_q = jnp.where(q1, _POS, s)
        q2, n2 = _pick_extreme(s_q, iota, False)
        eb = jnp.exp(n1 - n2)
        zb = 1.0 + eb
        wb_s[...] = jnp.where(q1, eb / zb, 0.0) + jnp.where(q2, 1.0 / zb, 0.0)
        top_ref[...] = jnp.zeros_like(top_ref)
        bot_ref[...] = jnp.zeros_like(bot_ref)

    xb = x_ref[...].astype(jnp.bfloat16)
    h = jax.lax.dot_general(
        xb, w1_ref[0].astype(jnp.bfloat16), (((1,), (1,)), ((), ())),
        preferred_element_type=jnp.float32) + b1_ref[0]
    h = _layer_norm(h, g1_ref[0], be1_ref[0])
    h = jnp.maximum(h, 0.0).astype(jnp.bfloat16)
    o = jax.lax.dot_general(
        h, w2_ref[0].astype(jnp.bfloat16), (((1,), (1,)), ((), ())),
        preferred_element_type=jnp.float32) + b2_ref[0]
    o = _layer_norm(o, g2_ref[0], be2_ref[0])

    lane = jax.lax.broadcasted_iota(jnp.int32, (BT, E), 1)
    sel = lane == e
    wt_col = jnp.sum(jnp.where(sel, wt_s[...], 0.0), axis=1, keepdims=True)
    wb_col = jnp.sum(jnp.where(sel, wb_s[...], 0.0), axis=1, keepdims=True)
    top_ref[...] += wt_col * o
    bot_ref[...] += wb_col * o

    @pl.when(e == E - 1)
    def _emit():
        at = top_ref[...]
        ab = bot_ref[...]
        out_ref[...] = at + x_ref[...]
        d = at - ab
        ss_ref[...] = jnp.full(ss_ref.shape, jnp.sum(d * d), jnp.float32)


def _moe_fused(xf, Wg, bg, W1, b1, g1, be1, W2, b2, g2, be2, *, BT):
    T, D = xf.shape
    E = Wg.shape[0]
    ntb = T // BT
    grid = (ntb, E)

    b1r = b1.reshape(E, 1, D)
    g1r = g1.reshape(E, 1, D)
    be1r = be1.reshape(E, 1, D)
    b2r = b2.reshape(E, 1, D)
    g2r = g2.reshape(E, 1, D)
    be2r = be2.reshape(E, 1, D)
    bgr = bg.reshape(1, E)

    def tb_map(tb, e):
        return (tb, 0)

    def e3_map(tb, e):
        return (e, 0, 0)

    out, top, bot, ss = pl.pallas_call(
        functools.partial(_moe_body, E, BT),
        grid=grid,
        in_specs=[
            pl.BlockSpec((BT, D), tb_map),
            pl.BlockSpec((E, D), lambda tb, e: (0, 0)),
            pl.BlockSpec((1, E), lambda tb, e: (0, 0)),
            pl.BlockSpec((1, D, D), e3_map),
            pl.BlockSpec((1, 1, D), e3_map),
            pl.BlockSpec((1, 1, D), e3_map),
            pl.BlockSpec((1, 1, D), e3_map),
            pl.BlockSpec((1, D, D), e3_map),
            pl.BlockSpec((1, 1, D), e3_map),
            pl.BlockSpec((1, 1, D), e3_map),
            pl.BlockSpec((1, 1, D), e3_map),
        ],
        out_specs=[
            pl.BlockSpec((BT, D), tb_map),
            pl.BlockSpec((BT, D), tb_map),
            pl.BlockSpec((BT, D), tb_map),
            pl.BlockSpec((8, 128), tb_map),
        ],
        out_shape=[
            jax.ShapeDtypeStruct((T, D), jnp.float32),
            jax.ShapeDtypeStruct((T, D), jnp.float32),
            jax.ShapeDtypeStruct((T, D), jnp.float32),
            jax.ShapeDtypeStruct((ntb * 8, 128), jnp.float32),
        ],
        scratch_shapes=[
            pltpu.VMEM((BT, E), jnp.float32),
            pltpu.VMEM((BT, E), jnp.float32),
        ],
    )(xf, Wg, bgr, W1, b1r, g1r, be1r, W2, b2r, g2r, be2r)
    return out, top, bot, ss


def kernel(x, Wg, bg, W1, b1, g1, be1, W2, b2, g2, be2):
    B_, N_, D_ = x.shape
    T = B_ * N_
    xf = x.reshape(T, D_)
    BT = min(1024, T)
    out, top, bot, ss = _moe_fused(
        xf, Wg, bg, W1, b1, g1, be1, W2, b2, g2, be2, BT=BT)
    total_ss = jnp.sum(ss[::8, 0])
    dist = jnp.sqrt(total_ss)
    loss = jnp.mean(1.0 / (dist + 1e-8))
    return (out.reshape(B_, N_, D_),
            top.reshape(B_, N_, D_),
            bot.reshape(B_, N_, D_),
            loss)
